# baseline (speedup 1.0000x reference)
import functools

import jax
import jax.numpy as jnp
from jax import lax
from jax.experimental import pallas as pl
from jax.experimental.pallas import tpu as pltpu
from jax.experimental.pallas import tpu_sc as plsc

NC, NS = 2, 16
NW = NC * NS
CH = 125
PR = 2 * CH
DEGW = 16
F32 = jnp.float32


def _sc_gather(x, idx2d):
    n, d = x.shape
    e = idx2d.size
    n_outer = (e // 2) // (NW * PR)
    half_rows = e // 2 // CH
    mesh = plsc.VectorSubcoreMesh(core_axis_name="c", subcore_axis_name="s",
                                  num_cores=NC, num_subcores=NS)

    @functools.partial(
        pl.kernel,
        out_type=jax.ShapeDtypeStruct((e // 2, 2 * d), F32),
        mesh=mesh,
        scratch_types=[
            pltpu.VMEM((4, CH), jnp.int32),
            pltpu.VMEM((PR, d), F32),
            pltpu.VMEM((PR, d), F32),
            pltpu.SemaphoreType.DMA,
        ],
        compiler_params=pltpu.CompilerParams(use_tc_tiling_on_sc=False),
    )
    def k(x_hbm, idx_hbm, out_hbm, idx_v, buf_l, buf_r, sem):
        wid = lax.axis_index("s") * NC + lax.axis_index("c")
        row0 = wid * (n_outer * 2)

        def outer(o, carry):
            pltpu.sync_copy(idx_hbm.at[pl.ds(row0 + o * 2, 2)],
                            idx_v.at[pl.ds(0, 2)])
            pltpu.sync_copy(idx_hbm.at[pl.ds(half_rows + row0 + o * 2, 2)],
                            idx_v.at[pl.ds(2, 2)])
            descs = []
            for j in range(2):
                descs.append(pltpu.async_copy(
                    x_hbm.at[idx_v.at[j]],
                    buf_l.at[pl.ds(j * CH, CH)], sem))
            for j in range(2):
                descs.append(pltpu.async_copy(
                    x_hbm.at[idx_v.at[2 + j]],
                    buf_r.at[pl.ds(j * CH, CH)], sem))
            for d_ in descs:
                d_.wait()
            base = (row0 + o * 2) * CH
            pltpu.sync_copy(buf_l, out_hbm.at[pl.ds(base, PR), pl.ds(0, d)])
            pltpu.sync_copy(buf_r, out_hbm.at[pl.ds(base, PR), pl.ds(d, d)])
            return carry

        lax.fori_loop(0, n_outer, outer, 0)

    return k(x, idx2d)


def _sc_scatter(zgp, col2d, n):
    ep, d2 = zgp.shape
    d = d2 // 2
    e = ep * 2
    n_outer = ep // (NW * PR)
    half_rows = ep // CH
    rows_per_tile = n // NS
    mesh = plsc.VectorSubcoreMesh(core_axis_name="c", subcore_axis_name="s",
                                  num_cores=NC, num_subcores=NS)

    @functools.partial(
        pl.kernel,
        out_type=(jax.ShapeDtypeStruct((NC, n, d), F32),
                  jax.ShapeDtypeStruct((NC, n, DEGW), F32)),
        mesh=mesh,
        scratch_types=[
            pltpu.VMEM((4, CH), jnp.int32),
            pltpu.VMEM((PR, d), F32),
            pltpu.VMEM((PR, d), F32),
            pltpu.VMEM((CH, DEGW), F32),
            pltpu.VMEM_SHARED((n, d), F32),
            pltpu.VMEM_SHARED((n, DEGW), F32),
        ],
        compiler_params=pltpu.CompilerParams(use_tc_tiling_on_sc=False),
    )
    def k(zg_hbm, col_hbm, out_hbm, deg_hbm, col_v, buf_l, buf_r, ones_v,
          acc, dacc):
        cid = lax.axis_index("c")
        sid = lax.axis_index("s")
        wid = cid * NS + sid

        def orow(i, carry):
            ones_v[i, pl.ds(0, DEGW)] = jnp.ones((DEGW,), F32)
            return carry
        lax.fori_loop(0, CH, orow, 0)

        def zrow(i, carry):
            for j in range(d // 16):
                buf_l[i, pl.ds(j * 16, 16)] = jnp.zeros((16,), F32)
            return carry
        lax.fori_loop(0, CH, zrow, 0)
        for r in range(rows_per_tile // CH):
            pltpu.sync_copy(buf_l.at[pl.ds(0, CH)],
                            acc.at[pl.ds(sid * rows_per_tile + r * CH, CH)])
            pltpu.sync_copy(buf_l.at[pl.ds(0, CH), pl.ds(0, DEGW)],
                            dacc.at[pl.ds(sid * rows_per_tile + r * CH, CH)])
        plsc.subcore_barrier()

        row0 = wid * (n_outer * 2)

        def outer(o, carry):
            pltpu.sync_copy(col_hbm.at[pl.ds(row0 + o * 2, 2)],
                            col_v.at[pl.ds(0, 2)])
            pltpu.sync_copy(col_hbm.at[pl.ds(half_rows + row0 + o * 2, 2)],
                            col_v.at[pl.ds(2, 2)])
            base = (row0 + o * 2) * CH
            pltpu.sync_copy(zg_hbm.at[pl.ds(base, PR), pl.ds(0, d)], buf_l)
            pltpu.sync_copy(zg_hbm.at[pl.ds(base, PR), pl.ds(d, d)], buf_r)
            for j in range(2):
                pltpu.sync_copy(buf_l.at[pl.ds(j * CH, CH)],
                                acc.at[col_v.at[j]], add=True)
                pltpu.sync_copy(ones_v, dacc.at[col_v.at[j]], add=True)
            for j in range(2):
                pltpu.sync_copy(buf_r.at[pl.ds(j * CH, CH)],
                                acc.at[col_v.at[2 + j]], add=True)
                pltpu.sync_copy(ones_v, dacc.at[col_v.at[2 + j]], add=True)
            return carry

        lax.fori_loop(0, n_outer, outer, 0)
        plsc.subcore_barrier()
        pltpu.sync_copy(acc.at[pl.ds(sid * rows_per_tile, rows_per_tile)],
                        out_hbm.at[cid, pl.ds(sid * rows_per_tile, rows_per_tile)])
        pltpu.sync_copy(dacc.at[pl.ds(sid * rows_per_tile, rows_per_tile)],
                        deg_hbm.at[cid, pl.ds(sid * rows_per_tile, rows_per_tile)])

    return k(zgp, col2d)


def _dot(a, b):
    return jnp.dot(a, b, preferred_element_type=F32)


def _full(arr):
    return pl.BlockSpec(arr.shape, lambda i: (0,) * arr.ndim)


def _pre_w0(x, w0x, blk):
    n, h = x.shape
    d = w0x.shape[1]

    def body(x_r, w_r, o_r):
        o_r[...] = _dot(x_r[...], w_r[...])

    return pl.pallas_call(
        body,
        grid=(n // blk,),
        in_specs=[pl.BlockSpec((blk, h), lambda i: (i, 0)), _full(w0x)],
        out_specs=pl.BlockSpec((blk, d), lambda i: (i, 0)),
        out_shape=jax.ShapeDtypeStruct((n, d), F32),
    )(x, w0x)


def _edge_mlp(xgp, ea, w0e, b0, w1, b1, w2, b2, g, blk):
    ep, d2 = xgp.shape
    d = d2 // 2
    e, h = ea.shape
    nblk = ep // blk

    def half(xh, eah, w0e_v, b0_v, w1_v, b1_v, w2_v, b2_v, g_v):
        hh = jnp.maximum(xh + _dot(eah, w0e_v) + b0_v, 0.0)
        hh = jnp.maximum(_dot(hh, w1_v) + b1_v, 0.0)
        hh = jnp.maximum(_dot(hh, w2_v) + b2_v, 0.0)
        mu = jnp.mean(hh, axis=-1, keepdims=True)
        var = jnp.mean(jnp.square(hh - mu), axis=-1, keepdims=True)
        return (hh - mu) * lax.rsqrt(var + 1e-5) * g_v

    def body(xg_r, ealo_r, eahi_r, w0e_r, b0_r, w1_r, b1_r, w2_r, b2_r, g_r,
             o_r):
        xg = xg_r[...]
        args = (w0e_r[...], b0_r[...], w1_r[...], b1_r[...], w2_r[...],
                b2_r[...], g_r[...])
        zl = half(xg[:, :d], ealo_r[...], *args)
        zr = half(xg[:, d:], eahi_r[...], *args)
        o_r[...] = jnp.concatenate([zl, zr], axis=-1)

    return pl.pallas_call(
        body,
        grid=(nblk,),
        in_specs=[
            pl.BlockSpec((blk, d2), lambda i: (i, 0)),
            pl.BlockSpec((blk, h), lambda i: (i, 0)),
            pl.BlockSpec((blk, h), lambda i, nb=nblk: (i + nb, 0)),
            _full(w0e), _full(b0), _full(w1), _full(b1), _full(w2), _full(b2),
            _full(g),
        ],
        out_specs=pl.BlockSpec((blk, d2), lambda i: (i, 0)),
        out_shape=jax.ShapeDtypeStruct((ep, d2), F32),
    )(xgp, ea, ea, w0e, b0, w1, b1, w2, b2, g)


def _node_mlp(x, parts, degs, u, batch2, w0a, w3w0b, degw0b, w0c, b0, w1, b1,
              w2, b2, g, beta, w3, b3, blk):
    n, h = x.shape
    nb = u.shape[0]
    d = parts.shape[-1]

    def body(x_r, p_r, dg_r, u_r, bt_r, w0a_r, w3w0b_r, degw0b_r, w0c_r, b0_r,
             w1_r, b1_r, w2_r, b2_r, g_r, beta_r, w3_r, b3_r, o_r):
        xv = x_r[...]
        p = p_r[...]
        pp = p[0] + p[1]
        dg = dg_r[...]
        dd = dg[0] + dg[1]
        bt = bt_r[...]
        oh = (bt == lax.broadcasted_iota(jnp.int32, (blk, nb), 1)).astype(F32)
        ub = _dot(oh, u_r[...])
        hh = jnp.maximum(_dot(xv, w0a_r[...]) + _dot(pp, w3w0b_r[...])
                         + _dot(dd, degw0b_r[...]) + _dot(ub, w0c_r[...])
                         + b0_r[...], 0.0)
        hh = jnp.maximum(_dot(hh, w1_r[...]) + b1_r[...], 0.0)
        hh = jnp.maximum(_dot(hh, w2_r[...]) + b2_r[...], 0.0)
        mu = jnp.mean(hh, axis=-1, keepdims=True)
        var = jnp.mean(jnp.square(hh - mu), axis=-1, keepdims=True)
        hh = (hh - mu) * lax.rsqrt(var + 1e-5) * g_r[...] + beta_r[...]
        o_r[...] = xv + _dot(hh, w3_r[...]) + b3_r[...]

    return pl.pallas_call(
        body,
        grid=(n // blk,),
        in_specs=[
            pl.BlockSpec((blk, h), lambda i: (i, 0)),
            pl.BlockSpec((NC, blk, d), lambda i: (0, i, 0)),
            pl.BlockSpec((NC, blk, DEGW), lambda i: (0, i, 0)),
            _full(u),
            pl.BlockSpec((blk, 1), lambda i: (i, 0)),
            _full(w0a), _full(w3w0b), _full(degw0b), _full(w0c), _full(b0),
            _full(w1), _full(b1), _full(w2), _full(b2), _full(g), _full(beta),
            _full(w3), _full(b3),
        ],
        out_specs=pl.BlockSpec((blk, h), lambda i: (i, 0)),
        out_shape=jax.ShapeDtypeStruct((n, h), F32),
    )(x, parts, degs, u, batch2, w0a, w3w0b, degw0b, w0c, b0, w1, b1, w2, b2,
      g, beta, w3, b3)


def kernel(x, edge_index, edge_attr, u, batch,
           m1_W0, m1_b0, m1_W1, m1_b1, m1_W2, m1_b2, m1_g, m1_beta, m1_W3, m1_b3,
           m2_W0, m2_b0, m2_W1, m2_b1, m2_W2, m2_b2, m2_g, m2_beta, m2_W3, m2_b3):
    n, h = x.shape
    e = edge_attr.shape[0]
    row2 = edge_index[0].reshape(e // CH, CH)
    col2 = edge_index[1].reshape(e // CH, CH)

    xw = _pre_w0(x, m1_W0[:h], blk=2000)
    xgp = _sc_gather(xw, row2)

    zgp = _edge_mlp(xgp, edge_attr, m1_W0[h:], m1_b0.reshape(1, -1),
                    m1_W1, m1_b1.reshape(1, -1), m1_W2, m1_b2.reshape(1, -1),
                    m1_g.reshape(1, -1), blk=10000)

    parts, degs = _sc_scatter(zgp, col2, n)

    w0b = m2_W0[h:2 * h]
    w3w0b = m1_W3 @ w0b
    cvec = (m1_beta @ m1_W3 + m1_b3) @ w0b
    degw0b = jnp.ones((DEGW, 1), F32) @ cvec.reshape(1, -1) / DEGW

    out = _node_mlp(x, parts, degs, u, batch.reshape(n, 1),
                    m2_W0[:h], w3w0b, degw0b, m2_W0[2 * h:],
                    m2_b0.reshape(1, -1), m2_W1, m2_b1.reshape(1, -1),
                    m2_W2, m2_b2.reshape(1, -1), m2_g.reshape(1, -1),
                    m2_beta.reshape(1, -1), m2_W3, m2_b3.reshape(1, -1),
                    blk=2000)
    return out


# speedup vs baseline: 1.1445x; 1.1445x over previous
import functools

import jax
import jax.numpy as jnp
from jax import lax
from jax.experimental import pallas as pl
from jax.experimental.pallas import tpu as pltpu
from jax.experimental.pallas import tpu_sc as plsc

NC, NS = 2, 16
NW = NC * NS
CH = 125
PR = 2 * CH
DEGW = 16
F32 = jnp.float32


def _sc_gather(x, idx2d, rb, half_rows):
    n, d = x.shape
    n_outer = (half_rows * CH) // (NW * PR)
    mesh = plsc.VectorSubcoreMesh(core_axis_name="c", subcore_axis_name="s",
                                  num_cores=NC, num_subcores=NS)

    @functools.partial(
        pl.kernel,
        out_type=jax.ShapeDtypeStruct((half_rows * CH, 2 * d), F32),
        mesh=mesh,
        scratch_types=[
            pltpu.VMEM((4, CH), jnp.int32),
            pltpu.VMEM((PR, d), F32),
            pltpu.VMEM((PR, d), F32),
            pltpu.SemaphoreType.DMA,
        ],
        compiler_params=pltpu.CompilerParams(use_tc_tiling_on_sc=False),
    )
    def k(x_hbm, idx_hbm, out_hbm, idx_v, buf_l, buf_r, sem):
        wid = lax.axis_index("s") * NC + lax.axis_index("c")
        row0 = wid * (n_outer * 2)

        def outer(o, carry):
            pltpu.sync_copy(idx_hbm.at[pl.ds(rb + row0 + o * 2, 2)],
                            idx_v.at[pl.ds(0, 2)])
            pltpu.sync_copy(
                idx_hbm.at[pl.ds(rb + half_rows + row0 + o * 2, 2)],
                idx_v.at[pl.ds(2, 2)])
            descs = []
            for j in range(2):
                descs.append(pltpu.async_copy(
                    x_hbm.at[idx_v.at[j]],
                    buf_l.at[pl.ds(j * CH, CH)], sem))
            for j in range(2):
                descs.append(pltpu.async_copy(
                    x_hbm.at[idx_v.at[2 + j]],
                    buf_r.at[pl.ds(j * CH, CH)], sem))
            for d_ in descs:
                d_.wait()
            base = (row0 + o * 2) * CH
            pltpu.sync_copy(buf_l, out_hbm.at[pl.ds(base, PR), pl.ds(0, d)])
            pltpu.sync_copy(buf_r, out_hbm.at[pl.ds(base, PR), pl.ds(d, d)])
            return carry

        lax.fori_loop(0, n_outer, outer, 0)

    return k(x, idx2d)


def _sc_scatter(zgp, col2d, n, rb):
    ep, d2 = zgp.shape
    d = d2 // 2
    n_outer = ep // (NW * PR)
    half_rows = ep // CH
    rows_per_tile = n // NS
    mesh = plsc.VectorSubcoreMesh(core_axis_name="c", subcore_axis_name="s",
                                  num_cores=NC, num_subcores=NS)

    @functools.partial(
        pl.kernel,
        out_type=(jax.ShapeDtypeStruct((NC, n, d), F32),
                  jax.ShapeDtypeStruct((NC, n, DEGW), F32)),
        mesh=mesh,
        scratch_types=[
            pltpu.VMEM((4, CH), jnp.int32),
            pltpu.VMEM((PR, d), F32),
            pltpu.VMEM((PR, d), F32),
            pltpu.VMEM((CH, DEGW), F32),
            pltpu.VMEM_SHARED((n, d), F32),
            pltpu.VMEM_SHARED((n, DEGW), F32),
        ],
        compiler_params=pltpu.CompilerParams(use_tc_tiling_on_sc=False),
    )
    def k(zg_hbm, col_hbm, out_hbm, deg_hbm, col_v, buf_l, buf_r, ones_v,
          acc, dacc):
        cid = lax.axis_index("c")
        sid = lax.axis_index("s")
        wid = cid * NS + sid

        def orow(i, carry):
            ones_v[i, pl.ds(0, DEGW)] = jnp.ones((DEGW,), F32)
            return carry
        lax.fori_loop(0, CH, orow, 0)

        def zrow(i, carry):
            for j in range(d // 16):
                buf_l[i, pl.ds(j * 16, 16)] = jnp.zeros((16,), F32)
            return carry
        lax.fori_loop(0, CH, zrow, 0)
        for r in range(rows_per_tile // CH):
            pltpu.sync_copy(buf_l.at[pl.ds(0, CH)],
                            acc.at[pl.ds(sid * rows_per_tile + r * CH, CH)])
            pltpu.sync_copy(buf_l.at[pl.ds(0, CH), pl.ds(0, DEGW)],
                            dacc.at[pl.ds(sid * rows_per_tile + r * CH, CH)])
        plsc.subcore_barrier()

        row0 = wid * (n_outer * 2)

        def outer(o, carry):
            pltpu.sync_copy(col_hbm.at[pl.ds(rb + row0 + o * 2, 2)],
                            col_v.at[pl.ds(0, 2)])
            pltpu.sync_copy(
                col_hbm.at[pl.ds(rb + half_rows + row0 + o * 2, 2)],
                col_v.at[pl.ds(2, 2)])
            base = (row0 + o * 2) * CH
            pltpu.sync_copy(zg_hbm.at[pl.ds(base, PR), pl.ds(0, d)], buf_l)
            pltpu.sync_copy(zg_hbm.at[pl.ds(base, PR), pl.ds(d, d)], buf_r)
            for j in range(2):
                pltpu.sync_copy(buf_l.at[pl.ds(j * CH, CH)],
                                acc.at[col_v.at[j]], add=True)
                pltpu.sync_copy(ones_v, dacc.at[col_v.at[j]], add=True)
            for j in range(2):
                pltpu.sync_copy(buf_r.at[pl.ds(j * CH, CH)],
                                acc.at[col_v.at[2 + j]], add=True)
                pltpu.sync_copy(ones_v, dacc.at[col_v.at[2 + j]], add=True)
            return carry

        lax.fori_loop(0, n_outer, outer, 0)
        plsc.subcore_barrier()
        pltpu.sync_copy(acc.at[pl.ds(sid * rows_per_tile, rows_per_tile)],
                        out_hbm.at[cid, pl.ds(sid * rows_per_tile, rows_per_tile)])
        pltpu.sync_copy(dacc.at[pl.ds(sid * rows_per_tile, rows_per_tile)],
                        deg_hbm.at[cid, pl.ds(sid * rows_per_tile, rows_per_tile)])

    return k(zgp, col2d)


def _dot(a, b):
    return jnp.dot(a, b, preferred_element_type=F32)


def _full(arr):
    return pl.BlockSpec(arr.shape, lambda i: (0,) * arr.ndim)


def _pre_w0(x, w0x, blk):
    n, h = x.shape
    d = w0x.shape[1]

    def body(x_r, w_r, o_r):
        o_r[...] = _dot(x_r[...], w_r[...])

    return pl.pallas_call(
        body,
        grid=(n // blk,),
        in_specs=[pl.BlockSpec((blk, h), lambda i: (i, 0)), _full(w0x)],
        out_specs=pl.BlockSpec((blk, d), lambda i: (i, 0)),
        out_shape=jax.ShapeDtypeStruct((n, d), F32),
    )(x, w0x)


def _edge_mlp(xgp, ea, w0e, b0, w1, b1, w2, b2, g, blk, eb):
    ep, d2 = xgp.shape
    d = d2 // 2
    e, h = ea.shape
    nblk = ep // blk

    def half(xh, eah, w0e_v, b0_v, w1_v, b1_v, w2_v, b2_v, g_v):
        hh = jnp.maximum(xh + _dot(eah, w0e_v) + b0_v, 0.0)
        hh = jnp.maximum(_dot(hh, w1_v) + b1_v, 0.0)
        hh = jnp.maximum(_dot(hh, w2_v) + b2_v, 0.0)
        mu = jnp.mean(hh, axis=-1, keepdims=True)
        var = jnp.mean(jnp.square(hh - mu), axis=-1, keepdims=True)
        return (hh - mu) * lax.rsqrt(var + 1e-5) * g_v

    def body(xg_r, ealo_r, eahi_r, w0e_r, b0_r, w1_r, b1_r, w2_r, b2_r, g_r,
             o_r):
        xg = xg_r[...]
        args = (w0e_r[...], b0_r[...], w1_r[...], b1_r[...], w2_r[...],
                b2_r[...], g_r[...])
        zl = half(xg[:, :d], ealo_r[...], *args)
        zr = half(xg[:, d:], eahi_r[...], *args)
        o_r[...] = jnp.concatenate([zl, zr], axis=-1)

    return pl.pallas_call(
        body,
        grid=(nblk,),
        in_specs=[
            pl.BlockSpec((blk, d2), lambda i: (i, 0)),
            pl.BlockSpec((blk, h), lambda i, b=eb: (i + b, 0)),
            pl.BlockSpec((blk, h), lambda i, b=eb + nblk: (i + b, 0)),
            _full(w0e), _full(b0), _full(w1), _full(b1), _full(w2), _full(b2),
            _full(g),
        ],
        out_specs=pl.BlockSpec((blk, d2), lambda i: (i, 0)),
        out_shape=jax.ShapeDtypeStruct((ep, d2), F32),
    )(xgp, ea, ea, w0e, b0, w1, b1, w2, b2, g)


def _node_mlp(x, parts, parts2, degs, degs2, u, batch2, w0a, w3w0b, degw0b,
              w0c, b0, w1, b1, w2, b2, g, beta, w3, b3, blk):
    n, h = x.shape
    nb = u.shape[0]
    d = parts.shape[-1]

    def body(x_r, p_r, q_r, dg_r, eh_r, u_r, bt_r, w0a_r, w3w0b_r, degw0b_r,
             w0c_r, b0_r, w1_r, b1_r, w2_r, b2_r, g_r, beta_r, w3_r, b3_r,
             o_r):
        xv = x_r[...]
        p = p_r[...]
        q = q_r[...]
        pp = p[0] + p[1] + q[0] + q[1]
        dg = dg_r[...]
        eh = eh_r[...]
        dd = dg[0] + dg[1] + eh[0] + eh[1]
        bt = bt_r[...]
        oh = (bt == lax.broadcasted_iota(jnp.int32, (blk, nb), 1)).astype(F32)
        ub = _dot(oh, u_r[...])
        hh = jnp.maximum(_dot(xv, w0a_r[...]) + _dot(pp, w3w0b_r[...])
                         + _dot(dd, degw0b_r[...]) + _dot(ub, w0c_r[...])
                         + b0_r[...], 0.0)
        hh = jnp.maximum(_dot(hh, w1_r[...]) + b1_r[...], 0.0)
        hh = jnp.maximum(_dot(hh, w2_r[...]) + b2_r[...], 0.0)
        mu = jnp.mean(hh, axis=-1, keepdims=True)
        var = jnp.mean(jnp.square(hh - mu), axis=-1, keepdims=True)
        hh = (hh - mu) * lax.rsqrt(var + 1e-5) * g_r[...] + beta_r[...]
        o_r[...] = xv + _dot(hh, w3_r[...]) + b3_r[...]

    return pl.pallas_call(
        body,
        grid=(n // blk,),
        in_specs=[
            pl.BlockSpec((blk, h), lambda i: (i, 0)),
            pl.BlockSpec((NC, blk, d), lambda i: (0, i, 0)),
            pl.BlockSpec((NC, blk, d), lambda i: (0, i, 0)),
            pl.BlockSpec((NC, blk, DEGW), lambda i: (0, i, 0)),
            pl.BlockSpec((NC, blk, DEGW), lambda i: (0, i, 0)),
            _full(u),
            pl.BlockSpec((blk, 1), lambda i: (i, 0)),
            _full(w0a), _full(w3w0b), _full(degw0b), _full(w0c), _full(b0),
            _full(w1), _full(b1), _full(w2), _full(b2), _full(g), _full(beta),
            _full(w3), _full(b3),
        ],
        out_specs=pl.BlockSpec((blk, h), lambda i: (i, 0)),
        out_shape=jax.ShapeDtypeStruct((n, h), F32),
    )(x, parts, parts2, degs, degs2, u, batch2, w0a, w3w0b, degw0b, w0c, b0,
      w1, b1, w2, b2, g, beta, w3, b3)


def kernel(x, edge_index, edge_attr, u, batch,
           m1_W0, m1_b0, m1_W1, m1_b1, m1_W2, m1_b2, m1_g, m1_beta, m1_W3, m1_b3,
           m2_W0, m2_b0, m2_W1, m2_b1, m2_W2, m2_b2, m2_g, m2_beta, m2_W3, m2_b3):
    n, h = x.shape
    e = edge_attr.shape[0]
    row2 = edge_index[0].reshape(e // CH, CH)
    col2 = edge_index[1].reshape(e // CH, CH)

    xw = _pre_w0(x, m1_W0[:h], blk=2000)

    qr = (e // CH) // 4
    blk_e = 8000
    m1_args = (m1_W0[h:], m1_b0.reshape(1, -1), m1_W1, m1_b1.reshape(1, -1),
               m1_W2, m1_b2.reshape(1, -1), m1_g.reshape(1, -1))
    xgA = _sc_gather(xw, row2, 0, qr)
    xgB = _sc_gather(xw, row2, 2 * qr, qr)
    zgA = _edge_mlp(xgA, edge_attr, *m1_args, blk=blk_e, eb=0)
    zgB = _edge_mlp(xgB, edge_attr, *m1_args, blk=blk_e,
                    eb=(e // 2) // blk_e)
    partsA, degsA = _sc_scatter(zgA, col2, n, 0)
    partsB, degsB = _sc_scatter(zgB, col2, n, 2 * qr)

    w0b = m2_W0[h:2 * h]
    w3w0b = m1_W3 @ w0b
    cvec = (m1_beta @ m1_W3 + m1_b3) @ w0b
    degw0b = jnp.ones((DEGW, 1), F32) @ cvec.reshape(1, -1) / DEGW

    out = _node_mlp(x, partsA, partsB, degsA, degsB, u, batch.reshape(n, 1),
                    m2_W0[:h], w3w0b, degw0b, m2_W0[2 * h:],
                    m2_b0.reshape(1, -1), m2_W1, m2_b1.reshape(1, -1),
                    m2_W2, m2_b2.reshape(1, -1), m2_g.reshape(1, -1),
                    m2_beta.reshape(1, -1), m2_W3, m2_b3.reshape(1, -1),
                    blk=2000)
    return out
